# baseline (speedup 1.0000x reference)
import functools

import jax
import jax.numpy as jnp
from jax import lax
from jax.experimental import pallas as pl
from jax.experimental.pallas import tpu as pltpu
from jax.experimental.pallas import tpu_sc as plsc

_REPLACE_ID = 100001

_B = 128
_L = 2048
_LANES = 16
_NC = 2
_NS = 16
_NW = _NC * _NS
_ROWS_PER_W = _B // _NW


def _prep_body(pml_ref, p_ref, sp_ref):
    x = pml_ref[...]
    p_ref[...] = 1.0 / (1.0 + jnp.exp(-x))
    sp_ref[...] = jnp.maximum(x, 0.0) + jnp.log1p(jnp.exp(-jnp.abs(x)))


def _sc_body(seq_hbm, u_hbm, p_hbm, pml_hbm, sp_hbm,
             seq_out_hbm, mask_hbm, logits_pad_hbm,
             p_v, pml_v, sp_v, seq_v, u_v, seq_out_v, mask_v, logits_v):
    wid = lax.axis_index("s") * _NC + lax.axis_index("c")
    base = wid * _ROWS_PER_W

    pltpu.sync_copy(p_hbm, p_v)
    pltpu.sync_copy(pml_hbm, pml_v)
    pltpu.sync_copy(sp_hbm, sp_v)
    pltpu.sync_copy(seq_hbm.at[pl.ds(base, _ROWS_PER_W)], seq_v)
    pltpu.sync_copy(u_hbm.at[pl.ds(base, _ROWS_PER_W)], u_v)

    for r in range(_ROWS_PER_W):
        def body(i, acc):
            off = i * _LANES
            sl = pl.ds(off, _LANES)
            m = u_v[r, sl] < p_v[sl]
            mask_v[r, sl] = jnp.where(m, 1.0, 0.0).astype(jnp.float32)
            seq_out_v[r, sl] = jnp.where(m, _REPLACE_ID, seq_v[r, sl])
            return acc + jnp.where(m, pml_v[sl], 0.0) - sp_v[sl]

        acc = lax.fori_loop(0, _L // _LANES, body,
                            jnp.zeros((_LANES,), jnp.float32))
        logits_v[r, :] = jnp.broadcast_to(jnp.sum(acc), (_LANES,))

    pltpu.sync_copy(seq_out_v, seq_out_hbm.at[pl.ds(base, _ROWS_PER_W)])
    pltpu.sync_copy(mask_v, mask_hbm.at[pl.ds(base, _ROWS_PER_W)])
    pltpu.sync_copy(logits_v, logits_pad_hbm.at[pl.ds(base, _ROWS_PER_W)])


@jax.jit
def kernel(sequence, prob_mask_logits, u):
    B, L = sequence.shape

    p2, sp2 = pl.pallas_call(
        _prep_body,
        out_shape=(
            jax.ShapeDtypeStruct((_LANES, L // _LANES), jnp.float32),
            jax.ShapeDtypeStruct((_LANES, L // _LANES), jnp.float32),
        ),
    )(prob_mask_logits.reshape(_LANES, L // _LANES))
    p = p2.reshape(L)
    sp = sp2.reshape(L)

    mesh = plsc.VectorSubcoreMesh(
        core_axis_name="c", subcore_axis_name="s",
        num_cores=_NC, num_subcores=_NS)

    sc = pl.kernel(
        _sc_body,
        out_type=(
            jax.ShapeDtypeStruct((B, L), jnp.int32),
            jax.ShapeDtypeStruct((B, L), jnp.float32),
            jax.ShapeDtypeStruct((B, _LANES), jnp.float32),
        ),
        mesh=mesh,
        compiler_params=pltpu.CompilerParams(needs_layout_passes=False),
        scratch_types=[
            pltpu.VMEM((L,), jnp.float32),
            pltpu.VMEM((L,), jnp.float32),
            pltpu.VMEM((L,), jnp.float32),
            pltpu.VMEM((_ROWS_PER_W, L), jnp.int32),
            pltpu.VMEM((_ROWS_PER_W, L), jnp.float32),
            pltpu.VMEM((_ROWS_PER_W, L), jnp.int32),
            pltpu.VMEM((_ROWS_PER_W, L), jnp.float32),
            pltpu.VMEM((_ROWS_PER_W, _LANES), jnp.float32),
        ],
    )
    seq_out, hard_mask, logits_pad = sc(sequence, u, p, prob_mask_logits, sp)
    return (seq_out, logits_pad[:, 0], hard_mask)


# speedup vs baseline: 1.1887x; 1.1887x over previous
import functools

import jax
import jax.numpy as jnp
from jax import lax
from jax.experimental import pallas as pl
from jax.experimental.pallas import tpu as pltpu
from jax.experimental.pallas import tpu_sc as plsc

_REPLACE_ID = 100001

_B = 128
_L = 2048
_LANES = 16
_NC = 2
_NS = 16
_NW = _NC * _NS
_ROWS_PER_W = _B // _NW


def _prep_body(pml_ref, p_ref, c_ref):
    x = pml_ref[...]
    p_ref[...] = 1.0 / (1.0 + jnp.exp(-x))
    sp = jnp.maximum(x, 0.0) + jnp.log1p(jnp.exp(-jnp.abs(x)))
    c_ref[...] = jnp.broadcast_to(jnp.sum(sp), (_LANES,))


def _sc_body(seq_hbm, u_hbm, p_hbm, pml_hbm, c_hbm,
             seq_out_hbm, mask_hbm, logits_pad_hbm,
             p_v, pml_v, c_v, seq_v, u_v, seq_out_v, mask_v, logits_v,
             in_sem, out_sem):
    wid = lax.axis_index("c") * _NS + lax.axis_index("s")
    base = wid * _ROWS_PER_W

    cp = pltpu.async_copy
    d1 = cp(p_hbm, p_v, in_sem)
    d2 = cp(pml_hbm, pml_v, in_sem)
    d3 = cp(c_hbm, c_v, in_sem)
    d4 = cp(seq_hbm.at[pl.ds(base, _ROWS_PER_W)], seq_v, in_sem)
    d5 = cp(u_hbm.at[pl.ds(base, _ROWS_PER_W)], u_v, in_sem)
    for d in (d1, d2, d3, d4, d5):
        d.wait()

    zero = jnp.zeros((_LANES,), jnp.float32)

    @plsc.parallel_loop(0, _L, step=_LANES, unroll=4, carry=(zero,) * _ROWS_PER_W)
    def accs(off, carry):
        sl = pl.ds(off, _LANES)
        pv = p_v[sl]
        lv = pml_v[sl]
        out = []
        for r in range(_ROWS_PER_W):
            m = u_v[r, sl] < pv
            mask_v[r, sl] = jnp.where(m, 1.0, 0.0).astype(jnp.float32)
            seq_out_v[r, sl] = jnp.where(m, _REPLACE_ID, seq_v[r, sl])
            out.append(carry[r] + jnp.where(m, lv, 0.0))
        return tuple(out)

    cvec = c_v[:]
    for r in range(_ROWS_PER_W):
        logits_v[r, :] = jnp.broadcast_to(jnp.sum(accs[r]), (_LANES,)) - cvec

    o1 = cp(seq_out_v, seq_out_hbm.at[pl.ds(base, _ROWS_PER_W)], out_sem)
    o2 = cp(mask_v, mask_hbm.at[pl.ds(base, _ROWS_PER_W)], out_sem)
    o3 = cp(logits_v, logits_pad_hbm.at[pl.ds(base, _ROWS_PER_W)], out_sem)
    for d in (o1, o2, o3):
        d.wait()


@jax.jit
def kernel(sequence, prob_mask_logits, u):
    B, L = sequence.shape

    p2, c = pl.pallas_call(
        _prep_body,
        out_shape=(
            jax.ShapeDtypeStruct((_LANES, L // _LANES), jnp.float32),
            jax.ShapeDtypeStruct((_LANES,), jnp.float32),
        ),
    )(prob_mask_logits.reshape(_LANES, L // _LANES))
    p = p2.reshape(L)

    mesh = plsc.VectorSubcoreMesh(
        core_axis_name="c", subcore_axis_name="s",
        num_cores=_NC, num_subcores=_NS)

    sc = pl.kernel(
        _sc_body,
        out_type=(
            jax.ShapeDtypeStruct((B, L), jnp.int32),
            jax.ShapeDtypeStruct((B, L), jnp.float32),
            jax.ShapeDtypeStruct((B, _LANES), jnp.float32),
        ),
        mesh=mesh,
        compiler_params=pltpu.CompilerParams(needs_layout_passes=False),
        scratch_types=[
            pltpu.VMEM((L,), jnp.float32),
            pltpu.VMEM((L,), jnp.float32),
            pltpu.VMEM((_LANES,), jnp.float32),
            pltpu.VMEM((_ROWS_PER_W, L), jnp.int32),
            pltpu.VMEM((_ROWS_PER_W, L), jnp.float32),
            pltpu.VMEM((_ROWS_PER_W, L), jnp.int32),
            pltpu.VMEM((_ROWS_PER_W, L), jnp.float32),
            pltpu.VMEM((_ROWS_PER_W, _LANES), jnp.float32),
            pltpu.SemaphoreType.DMA,
            pltpu.SemaphoreType.DMA,
        ],
    )
    seq_out, hard_mask, logits_pad = sc(sequence, u, p, prob_mask_logits, c)
    return (seq_out, logits_pad[:, 0], hard_mask)


# speedup vs baseline: 1.1940x; 1.0045x over previous
import functools

import jax
import jax.numpy as jnp
from jax import lax
from jax.experimental import pallas as pl
from jax.experimental.pallas import tpu as pltpu
from jax.experimental.pallas import tpu_sc as plsc

_REPLACE_ID = 100001

_B = 128
_L = 2048
_LANES = 16
_NC = 2
_NS = 16
_NW = _NC * _NS
_ROWS_PER_W = _B // _NW
_PAIR = 2


def _prep_body(pml_ref, p_ref, c_ref):
    x = pml_ref[...]
    p_ref[...] = 1.0 / (1.0 + jnp.exp(-x))
    sp = jnp.maximum(x, 0.0) + jnp.log1p(jnp.exp(-jnp.abs(x)))
    c_ref[...] = jnp.broadcast_to(jnp.sum(sp), (_LANES,))


def _sc_body(seq_hbm, u_hbm, p_hbm, pml_hbm, c_hbm,
             seq_out_hbm, mask_hbm, logits_hbm,
             p_v, pml_v, c_v,
             seq0_v, u0_v, so0_v, mk0_v,
             seq1_v, u1_v, so1_v, mk1_v,
             logits_v,
             sem_m, sem_0, sem_1, sem_o):
    cid = lax.axis_index("c")
    sid = lax.axis_index("s")
    wid = cid * _NS + sid
    base = wid * _ROWS_PER_W

    cp = pltpu.async_copy
    d_p = cp(p_hbm, p_v, sem_m)
    d_l = cp(pml_hbm, pml_v, sem_m)
    d_c = cp(c_hbm, c_v, sem_m)
    d_s0 = cp(seq_hbm.at[pl.ds(base, _PAIR)], seq0_v, sem_0)
    d_u0 = cp(u_hbm.at[pl.ds(base, _PAIR)], u0_v, sem_0)
    d_s1 = cp(seq_hbm.at[pl.ds(base + _PAIR, _PAIR)], seq1_v, sem_1)
    d_u1 = cp(u_hbm.at[pl.ds(base + _PAIR, _PAIR)], u1_v, sem_1)

    zero = jnp.zeros((_LANES,), jnp.float32)

    def run_pair(seq_v, u_v, so_v, mk_v):
        @plsc.parallel_loop(0, _L, step=_LANES, unroll=4,
                            carry=(zero,) * _PAIR)
        def accs(off, carry):
            sl = pl.ds(off, _LANES)
            pv = p_v[sl]
            lv = pml_v[sl]
            out = []
            for r in range(_PAIR):
                m = u_v[r, sl] < pv
                mk_v[r, sl] = jnp.where(m, 1.0, 0.0).astype(jnp.float32)
                so_v[r, sl] = jnp.where(m, _REPLACE_ID, seq_v[r, sl])
                out.append(carry[r] + jnp.where(m, lv, 0.0))
            return tuple(out)
        return accs

    d_p.wait()
    d_l.wait()
    d_s0.wait()
    d_u0.wait()
    accs0 = run_pair(seq0_v, u0_v, so0_v, mk0_v)
    o_s0 = cp(so0_v, seq_out_hbm.at[pl.ds(base, _PAIR)], sem_o)
    o_m0 = cp(mk0_v, mask_hbm.at[pl.ds(base, _PAIR)], sem_o)

    d_s1.wait()
    d_u1.wait()
    accs1 = run_pair(seq1_v, u1_v, so1_v, mk1_v)
    o_s1 = cp(so1_v, seq_out_hbm.at[pl.ds(base + _PAIR, _PAIR)], sem_o)
    o_m1 = cp(mk1_v, mask_hbm.at[pl.ds(base + _PAIR, _PAIR)], sem_o)

    d_c.wait()
    cvec = c_v[:]
    for r in range(_PAIR):
        logits_v[r, :] = jnp.broadcast_to(jnp.sum(accs0[r]), (_LANES,)) - cvec
        logits_v[_PAIR + r, :] = (
            jnp.broadcast_to(jnp.sum(accs1[r]), (_LANES,)) - cvec)

    pltpu.sync_copy(logits_v, logits_hbm.at[pl.ds(base, _ROWS_PER_W)])

    o_s0.wait()
    o_m0.wait()
    o_s1.wait()
    o_m1.wait()


@jax.jit
def kernel(sequence, prob_mask_logits, u):
    B, L = sequence.shape

    p2, c = pl.pallas_call(
        _prep_body,
        out_shape=(
            jax.ShapeDtypeStruct((_LANES, L // _LANES), jnp.float32),
            jax.ShapeDtypeStruct((_LANES,), jnp.float32),
        ),
    )(prob_mask_logits.reshape(_LANES, L // _LANES))
    p = p2.reshape(L)

    mesh = plsc.VectorSubcoreMesh(
        core_axis_name="c", subcore_axis_name="s",
        num_cores=_NC, num_subcores=_NS)

    sc = pl.kernel(
        _sc_body,
        out_type=(
            jax.ShapeDtypeStruct((B, L), jnp.int32),
            jax.ShapeDtypeStruct((B, L), jnp.float32),
            jax.ShapeDtypeStruct((B, _LANES), jnp.float32),
        ),
        mesh=mesh,
        compiler_params=pltpu.CompilerParams(needs_layout_passes=False),
        scratch_types=[
            pltpu.VMEM((L,), jnp.float32),
            pltpu.VMEM((L,), jnp.float32),
            pltpu.VMEM((_LANES,), jnp.float32),
            pltpu.VMEM((_PAIR, L), jnp.int32),
            pltpu.VMEM((_PAIR, L), jnp.float32),
            pltpu.VMEM((_PAIR, L), jnp.int32),
            pltpu.VMEM((_PAIR, L), jnp.float32),
            pltpu.VMEM((_PAIR, L), jnp.int32),
            pltpu.VMEM((_PAIR, L), jnp.float32),
            pltpu.VMEM((_PAIR, L), jnp.int32),
            pltpu.VMEM((_PAIR, L), jnp.float32),
            pltpu.VMEM((_ROWS_PER_W, _LANES), jnp.float32),
            pltpu.SemaphoreType.DMA,
            pltpu.SemaphoreType.DMA,
            pltpu.SemaphoreType.DMA,
            pltpu.SemaphoreType.DMA,
        ],
    )
    seq_out, hard_mask, logits_pad = sc(sequence, u, p, prob_mask_logits, c)
    return (seq_out, logits_pad[:, 0], hard_mask)
